# Initial kernel scaffold; baseline (speedup 1.0000x reference)
import jax
import jax.numpy as jnp
from jax.experimental import pallas as pl


def kernel(inputs, mask, mask_split):
    raise NotImplementedError("write your pallas kernel here")


# speedup vs baseline: 5.1906x; 5.1906x over previous
import functools

import jax
import jax.numpy as jnp
from jax import lax
from jax.experimental import pallas as pl
from jax.experimental.pallas import tpu as pltpu
from jax.experimental.pallas import tpu_sc as plsc

B = 16
L1 = 4096
MAXQ = 2048
MAXT = 512
MAXC = 1536
L2 = MAXQ + MAXT + MAXC
D = 512

CHUNK = 32
LANES = 16


def _repack_body(inputs_hbm, maski_hbm, out_hbm, mrow, ring, zbuf, mixbuf):
    cid = lax.axis_index("c")
    sid = lax.axis_index("s")
    wid = sid * 2 + cid
    b = wid // 2
    half = wid % 2

    in_base = b * (L1 * D)
    out_base = b * (L2 * D)

    zvec = jnp.zeros((LANES,), jnp.float32)

    def zb(j, _):
        zbuf[pl.ds(j * LANES, LANES)] = zvec
        return 0
    lax.fori_loop(0, (CHUNK * D) // LANES, zb, 0)

    pltpu.sync_copy(maski_hbm.at[pl.ds(b * L2, L2)], mrow)

    def msum(start, count):
        def sbody(i, acc):
            return acc + mrow[pl.ds(start + i * LANES, LANES)]
        acc = lax.fori_loop(0, count // LANES, sbody,
                            jnp.zeros((LANES,), jnp.int32))
        s = acc[0]
        for i in range(1, LANES):
            s = s + acc[i]
        return s

    def copy_seg(src_row, dst_row, n, region_rows):
        nfull = n // CHUNK
        rem = n - nfull * CHUNK
        nchunks = region_rows // CHUNK

        def cbody(g, _):
            src = in_base + (src_row + g * CHUNK) * D
            dst = out_base + (dst_row + g * CHUNK) * D
            pltpu.sync_copy(inputs_hbm.at[pl.ds(src, CHUNK * D)], ring)
            pltpu.sync_copy(ring, out_hbm.at[pl.ds(dst, CHUNK * D)])
            return 0
        lax.fori_loop(0, nfull, cbody, 0)

        first_zero = nfull + jnp.where(rem > 0, 1, 0)

        def zbody(g, _):
            dst = out_base + (dst_row + g * CHUNK) * D
            pltpu.sync_copy(zbuf, out_hbm.at[pl.ds(dst, CHUNK * D)])
            return 0
        lax.fori_loop(first_zero, nchunks, zbody, 0)

        @pl.when(rem > 0)
        def _():
            def zr(j, _):
                @pl.when(j >= rem * (D // LANES))
                def _():
                    mixbuf[pl.ds(j * LANES, LANES)] = zvec
                return 0
            lax.fori_loop(0, (CHUNK * D) // LANES, zr, 0)

            src0 = src_row + nfull * CHUNK
            def rrow(r, _):
                src = in_base + (src0 + r) * D
                pltpu.sync_copy(inputs_hbm.at[pl.ds(src, D)],
                                mixbuf.at[pl.ds(r * D, D)])
                return 0
            lax.fori_loop(0, rem, rrow, 0)
            dst = out_base + (dst_row + nfull * CHUNK) * D
            pltpu.sync_copy(mixbuf, out_hbm.at[pl.ds(dst, CHUNK * D)])

    q = msum(0, MAXQ)

    @pl.when(half == 0)
    def _():
        copy_seg(0, 0, q, MAXQ)

    @pl.when(half == 1)
    def _():
        t = msum(MAXQ, MAXT)
        c = msum(MAXQ + MAXT, MAXC)
        copy_seg(q, MAXQ, t, MAXT)
        copy_seg(q + t, MAXQ + MAXT, c, MAXC)


@jax.jit
def _repack(inputs, maski):
    mesh = plsc.VectorSubcoreMesh(core_axis_name="c", subcore_axis_name="s")
    k = functools.partial(
        pl.kernel,
        mesh=mesh,
        out_type=jax.ShapeDtypeStruct((B * L2 * D,), jnp.float32),
        scratch_types=[
            pltpu.VMEM((L2,), jnp.int32),
            pltpu.VMEM((CHUNK * D,), jnp.float32),
            pltpu.VMEM((CHUNK * D,), jnp.float32),
            pltpu.VMEM((CHUNK * D,), jnp.float32),
        ],
    )(_repack_body)
    out_flat = k(inputs.reshape(-1), maski.reshape(-1))
    return out_flat.reshape(B, L2, D)


def kernel(inputs, mask, mask_split):
    del mask
    outputs = _repack(inputs, mask_split.astype(jnp.int32))
    return outputs, mask_split


# speedup vs baseline: 5.7981x; 1.1170x over previous
import functools

import jax
import jax.numpy as jnp
from jax import lax
from jax.experimental import pallas as pl
from jax.experimental.pallas import tpu as pltpu
from jax.experimental.pallas import tpu_sc as plsc

B = 16
L1 = 4096
MAXQ = 2048
MAXT = 512
MAXC = 1536
L2 = MAXQ + MAXT + MAXC
D = 512

CHUNK = 32
LANES = 16


def _repack_body(inputs_hbm, maski_hbm, out_hbm,
                 mrow, ring0, ring1, zbuf, mixbuf,
                 rsem0, rsem1, wsem0, wsem1, zsem, msem):
    cid = lax.axis_index("c")
    sid = lax.axis_index("s")
    wid = sid * 2 + cid
    b = wid // 2
    half = wid % 2

    in_base = b * (L1 * D)
    out_base = b * (L2 * D)

    zvec = jnp.zeros((LANES,), jnp.float32)

    pltpu.make_async_copy(maski_hbm.at[pl.ds(b * L2, L2)], mrow, msem).start()

    def zb(j, _):
        zbuf[pl.ds(j * LANES, LANES)] = zvec
        return 0
    lax.fori_loop(0, (CHUNK * D) // LANES, zb, 0)

    pltpu.make_async_copy(maski_hbm.at[pl.ds(b * L2, L2)], mrow, msem).wait()

    def msum(start, count):
        def sbody(i, acc):
            return acc + mrow[pl.ds(start + i * LANES, LANES)]
        acc = lax.fori_loop(0, count // LANES, sbody,
                            jnp.zeros((LANES,), jnp.int32))
        s = acc[0]
        for i in range(1, LANES):
            s = s + acc[i]
        return s

    def copy_seg(src_row, dst_row, n, region_rows):
        nfull = n // CHUNK
        rem = n - nfull * CHUNK
        nchunks = region_rows // CHUNK

        def src_at(g):
            return inputs_hbm.at[pl.ds(in_base + (src_row + g * CHUNK) * D,
                                       CHUNK * D)]

        def dst_at(g):
            return out_hbm.at[pl.ds(out_base + (dst_row + g * CHUNK) * D,
                                    CHUNK * D)]

        first_zero = nfull + jnp.where(rem > 0, 1, 0)

        def zbody(g, _):
            pltpu.make_async_copy(zbuf, dst_at(g), zsem).start()
            return 0
        lax.fori_loop(first_zero, nchunks, zbody, 0)

        def pair(p, _):
            g0 = 2 * p
            g1 = g0 + 1

            @pl.when(p > 0)
            def _():
                pltpu.make_async_copy(ring0, dst_at(0), wsem0).wait()
            pltpu.make_async_copy(src_at(g0), ring0, rsem0).start()

            @pl.when(g1 < nfull)
            def _():
                @pl.when(p > 0)
                def _():
                    pltpu.make_async_copy(ring1, dst_at(0), wsem1).wait()
                pltpu.make_async_copy(src_at(g1), ring1, rsem1).start()

            pltpu.make_async_copy(src_at(g0), ring0, rsem0).wait()
            pltpu.make_async_copy(ring0, dst_at(g0), wsem0).start()

            @pl.when(g1 < nfull)
            def _():
                pltpu.make_async_copy(src_at(g1), ring1, rsem1).wait()
                pltpu.make_async_copy(ring1, dst_at(g1), wsem1).start()
            return 0
        lax.fori_loop(0, (nfull + 1) // 2, pair, 0)

        @pl.when(nfull >= 1)
        def _():
            pltpu.make_async_copy(ring0, dst_at(0), wsem0).wait()

        @pl.when(nfull >= 2)
        def _():
            pltpu.make_async_copy(ring1, dst_at(0), wsem1).wait()

        @pl.when(rem > 0)
        def _():
            def zr(j, _):
                @pl.when(j >= rem * (D // LANES))
                def _():
                    mixbuf[pl.ds(j * LANES, LANES)] = zvec
                return 0
            lax.fori_loop(0, (CHUNK * D) // LANES, zr, 0)

            src0 = src_row + nfull * CHUNK

            def rrow(r, _):
                src = in_base + (src0 + r) * D
                pltpu.make_async_copy(inputs_hbm.at[pl.ds(src, D)],
                                      mixbuf.at[pl.ds(r * D, D)],
                                      msem).start()
                return 0
            lax.fori_loop(0, rem, rrow, 0)

            def rdrain(r, _):
                pltpu.make_async_copy(inputs_hbm.at[pl.ds(in_base, D)],
                                      mixbuf.at[pl.ds(0, D)], msem).wait()
                return 0
            lax.fori_loop(0, rem, rdrain, 0)

            pltpu.make_async_copy(mixbuf, dst_at(nfull), zsem).start()

        nz = nchunks - first_zero + jnp.where(rem > 0, 1, 0)

        def zdrain(i, _):
            pltpu.make_async_copy(zbuf, dst_at(0), zsem).wait()
            return 0
        lax.fori_loop(0, nz, zdrain, 0)

    q = msum(0, MAXQ)

    @pl.when(half == 0)
    def _():
        copy_seg(0, 0, q, MAXQ)

    @pl.when(half == 1)
    def _():
        t = msum(MAXQ, MAXT)
        c = msum(MAXQ + MAXT, MAXC)
        copy_seg(q, MAXQ, t, MAXT)
        copy_seg(q + t, MAXQ + MAXT, c, MAXC)


@jax.jit
def _repack(inputs, maski):
    mesh = plsc.VectorSubcoreMesh(core_axis_name="c", subcore_axis_name="s")
    k = functools.partial(
        pl.kernel,
        mesh=mesh,
        out_type=jax.ShapeDtypeStruct((B * L2 * D,), jnp.float32),
        scratch_types=[
            pltpu.VMEM((L2,), jnp.int32),
            pltpu.VMEM((CHUNK * D,), jnp.float32),
            pltpu.VMEM((CHUNK * D,), jnp.float32),
            pltpu.VMEM((CHUNK * D,), jnp.float32),
            pltpu.VMEM((CHUNK * D,), jnp.float32),
            pltpu.SemaphoreType.DMA,
            pltpu.SemaphoreType.DMA,
            pltpu.SemaphoreType.DMA,
            pltpu.SemaphoreType.DMA,
            pltpu.SemaphoreType.DMA,
            pltpu.SemaphoreType.DMA,
        ],
    )(_repack_body)
    out_flat = k(inputs.reshape(-1), maski.reshape(-1))
    return out_flat.reshape(B, L2, D)


def kernel(inputs, mask, mask_split):
    del mask
    outputs = _repack(inputs, mask_split.astype(jnp.int32))
    return outputs, mask_split


# speedup vs baseline: 7.9589x; 1.3727x over previous
import functools

import jax
import jax.numpy as jnp
from jax import lax
from jax.experimental import pallas as pl
from jax.experimental.pallas import tpu as pltpu
from jax.experimental.pallas import tpu_sc as plsc

B = 16
L1 = 4096
MAXQ = 2048
MAXT = 512
MAXC = 1536
L2 = MAXQ + MAXT + MAXC
D = 512

CHUNK = 32
WIN = CHUNK + 8
LANES = 16


def _mult8(x):
    return pl.multiple_of(x, 8)


def _repack_body(inputs_hbm, maski_hbm, out_hbm,
                 mrow, ring0, ring1, obuf0, obuf1, zbuf,
                 rsem0, rsem1, wsem0, wsem1, zsem, msem):
    cid = lax.axis_index("c")
    sid = lax.axis_index("s")
    wid = sid * 2 + cid
    b = wid // 2
    half = wid % 2

    zvec = jnp.zeros((LANES,), jnp.float32)

    pltpu.make_async_copy(maski_hbm.at[pl.ds(b * L2, L2)], mrow, msem).start()

    def zb(j, _):
        zbuf[j // (D // LANES), pl.ds((j % (D // LANES)) * LANES, LANES)] = zvec
        return 0
    lax.fori_loop(0, (CHUNK * D) // LANES, zb, 0)

    pltpu.make_async_copy(maski_hbm.at[pl.ds(b * L2, L2)], mrow, msem).wait()

    def msum(start, count):
        def sbody(i, acc):
            return acc + mrow[pl.ds(start + i * LANES, LANES)]
        acc = lax.fori_loop(0, count // LANES, sbody,
                            jnp.zeros((LANES,), jnp.int32))
        s = acc[0]
        for i in range(1, LANES):
            s = s + acc[i]
        return s

    def shift_chunk(src_buf, sbase, nvalid, dst_buf):
        def srow(j, _):
            @pl.when(j < nvalid)
            def _():
                for ci in range(D // LANES):
                    dst_buf[j, pl.ds(ci * LANES, LANES)] = (
                        src_buf[sbase + j, pl.ds(ci * LANES, LANES)])
            @pl.when(j >= nvalid)
            def _():
                for ci in range(D // LANES):
                    dst_buf[j, pl.ds(ci * LANES, LANES)] = zvec
            return 0
        lax.fori_loop(0, CHUNK, srow, 0)

    def dst_at(dst_row, g):
        return out_hbm.at[b, pl.ds(_mult8(dst_row + g * CHUNK), CHUNK), :]

    def zero_tail(dst_row, nct, nchunks):
        def zbody(g, _):
            pltpu.make_async_copy(zbuf, dst_at(dst_row, g), zsem).start()
            return 0
        lax.fori_loop(nct, nchunks, zbody, 0)

    def seg_aligned(src_row, dst_row, n, region_rows):
        nfull = n // CHUNK
        rem = n - nfull * CHUNK
        nchunks = region_rows // CHUNK
        nct = nfull + jnp.where(rem > 0, 1, 0)
        zero_tail(dst_row, nct, nchunks)

        def src_at(g):
            return inputs_hbm.at[b, pl.ds(_mult8(src_row + g * CHUNK),
                                          CHUNK), :]

        r0 = ring0.at[pl.ds(0, CHUNK), :]
        r1 = ring1.at[pl.ds(0, CHUNK), :]

        def pair(p, _):
            g0 = 2 * p
            g1 = g0 + 1

            @pl.when(p > 0)
            def _():
                pltpu.make_async_copy(r0, dst_at(dst_row, 0), wsem0).wait()
            pltpu.make_async_copy(src_at(g0), r0, rsem0).start()

            @pl.when(g1 < nfull)
            def _():
                @pl.when(p > 0)
                def _():
                    pltpu.make_async_copy(r1, dst_at(dst_row, 0),
                                          wsem1).wait()
                pltpu.make_async_copy(src_at(g1), r1, rsem1).start()

            pltpu.make_async_copy(src_at(g0), r0, rsem0).wait()
            pltpu.make_async_copy(r0, dst_at(dst_row, g0), wsem0).start()

            @pl.when(g1 < nfull)
            def _():
                pltpu.make_async_copy(src_at(g1), r1, rsem1).wait()
                pltpu.make_async_copy(r1, dst_at(dst_row, g1), wsem1).start()
            return 0
        lax.fori_loop(0, (nfull + 1) // 2, pair, 0)

        @pl.when(nfull >= 1)
        def _():
            pltpu.make_async_copy(r0, dst_at(dst_row, 0), wsem0).wait()

        @pl.when(nfull >= 2)
        def _():
            pltpu.make_async_copy(r1, dst_at(dst_row, 0), wsem1).wait()

        @pl.when(rem > 0)
        def _():
            pltpu.make_async_copy(src_at(nfull), r0, rsem0).start()
            pltpu.make_async_copy(src_at(nfull), r0, rsem0).wait()
            shift_chunk(ring0, 0, rem, obuf0)
            pltpu.make_async_copy(obuf0, dst_at(dst_row, nfull), zsem).start()

        nz = nchunks - nct + jnp.where(rem > 0, 1, 0)

        def zdrain(i, _):
            pltpu.make_async_copy(zbuf, dst_at(dst_row, 0), zsem).wait()
            return 0
        lax.fori_loop(0, nz, zdrain, 0)

    def seg_shifted(src_row, dst_row, n, region_rows):
        nchunks = region_rows // CHUNK
        nct = (n + CHUNK - 1) // CHUNK
        zero_tail(dst_row, nct, nchunks)

        def win(g):
            src = src_row + g * CHUNK
            wstart = _mult8(jnp.minimum(src - src % 8, L1 - WIN))
            return inputs_hbm.at[b, pl.ds(wstart, WIN), :], src - wstart

        def proc(g, ring, obuf, rsem, wsem, first):
            _, sbase = win(g)
            nv = jnp.minimum(n - g * CHUNK, CHUNK)
            pltpu.make_async_copy(win(g)[0], ring, rsem).wait()
            @pl.when(jnp.logical_not(first))
            def _():
                pltpu.make_async_copy(obuf, dst_at(dst_row, 0), wsem).wait()
            shift_chunk(ring, sbase, nv, obuf)
            pltpu.make_async_copy(obuf, dst_at(dst_row, g), wsem).start()

        @pl.when(nct > 0)
        def _():
            pltpu.make_async_copy(win(0)[0], ring0, rsem0).start()

        def pair(p, _):
            g0 = 2 * p
            g1 = g0 + 1

            @pl.when(g1 < nct)
            def _():
                pltpu.make_async_copy(win(g1)[0], ring1, rsem1).start()

            proc(g0, ring0, obuf0, rsem0, wsem0, p == 0)

            @pl.when(g0 + 2 < nct)
            def _():
                pltpu.make_async_copy(win(g0 + 2)[0], ring0, rsem0).start()

            @pl.when(g1 < nct)
            def _():
                proc(g1, ring1, obuf1, rsem1, wsem1, p == 0)
            return 0
        lax.fori_loop(0, (nct + 1) // 2, pair, 0)

        @pl.when(nct >= 1)
        def _():
            pltpu.make_async_copy(obuf0, dst_at(dst_row, 0), wsem0).wait()

        @pl.when(nct >= 2)
        def _():
            pltpu.make_async_copy(obuf1, dst_at(dst_row, 0), wsem1).wait()

        nz = nchunks - nct

        def zdrain(i, _):
            pltpu.make_async_copy(zbuf, dst_at(dst_row, 0), zsem).wait()
            return 0
        lax.fori_loop(0, nz, zdrain, 0)

    q = msum(0, MAXQ)

    @pl.when(half == 0)
    def _():
        seg_aligned(0, 0, q, MAXQ)

    @pl.when(half == 1)
    def _():
        t = msum(MAXQ, MAXT)
        c = msum(MAXQ + MAXT, MAXC)
        seg_shifted(q, MAXQ, t, MAXT)
        seg_shifted(q + t, MAXQ + MAXT, c, MAXC)


@jax.jit
def _repack(inputs, maski):
    mesh = plsc.VectorSubcoreMesh(core_axis_name="c", subcore_axis_name="s")
    k = functools.partial(
        pl.kernel,
        mesh=mesh,
        out_type=jax.ShapeDtypeStruct((B, L2, D), jnp.float32),
        scratch_types=[
            pltpu.VMEM((L2,), jnp.int32),
            pltpu.VMEM((WIN, D), jnp.float32),
            pltpu.VMEM((WIN, D), jnp.float32),
            pltpu.VMEM((CHUNK, D), jnp.float32),
            pltpu.VMEM((CHUNK, D), jnp.float32),
            pltpu.VMEM((CHUNK, D), jnp.float32),
            pltpu.SemaphoreType.DMA,
            pltpu.SemaphoreType.DMA,
            pltpu.SemaphoreType.DMA,
            pltpu.SemaphoreType.DMA,
            pltpu.SemaphoreType.DMA,
            pltpu.SemaphoreType.DMA,
        ],
    )(_repack_body)
    return k(inputs, maski)


def kernel(inputs, mask, mask_split):
    del mask
    outputs = _repack(inputs, mask_split.astype(jnp.int32).reshape(-1))
    return outputs, mask_split


# speedup vs baseline: 11.1124x; 1.3962x over previous
import functools

import jax
import jax.numpy as jnp
from jax import lax
from jax.experimental import pallas as pl
from jax.experimental.pallas import tpu as pltpu
from jax.experimental.pallas import tpu_sc as plsc

B = 16
L1 = 4096
MAXQ = 2048
MAXT = 512
MAXC = 1536
L2 = MAXQ + MAXT + MAXC
D = 512

CHUNK = 32
WIN = CHUNK + 8
LANES = 16


def _mult8(x):
    return pl.multiple_of(x, 8)


def _repack_body(inputs_hbm, maski_hbm, out_hbm,
                 mrow, ring0, ring1, obuf0, obuf1, zbuf,
                 rsem0, rsem1, wsem0, wsem1, zsem, msem):
    cid = lax.axis_index("c")
    sid = lax.axis_index("s")
    wid = sid * 2 + cid
    b = wid // 2
    half = wid % 2

    zvec = jnp.zeros((LANES,), jnp.float32)

    pltpu.make_async_copy(maski_hbm.at[pl.ds(b * L2, L2)], mrow, msem).start()

    def zb(j, _):
        zbuf[j // (D // LANES), pl.ds((j % (D // LANES)) * LANES, LANES)] = zvec
        return 0
    lax.fori_loop(0, (CHUNK * D) // LANES, zb, 0)

    pltpu.make_async_copy(maski_hbm.at[pl.ds(b * L2, L2)], mrow, msem).wait()

    def msum(start, count):
        def sbody(i, acc):
            return acc + mrow[pl.ds(start + i * LANES, LANES)]
        acc = lax.fori_loop(0, count // LANES, sbody,
                            jnp.zeros((LANES,), jnp.int32))
        s = acc[0]
        for i in range(1, LANES):
            s = s + acc[i]
        return s

    def shift_chunk(src_buf, sbase, nvalid, dst_buf):
        def crow(j, _):
            for ci in range(D // LANES):
                dst_buf[j, pl.ds(ci * LANES, LANES)] = (
                    src_buf[sbase + j, pl.ds(ci * LANES, LANES)])
            return 0
        lax.fori_loop(0, nvalid, crow, 0)

        def zrow(j, _):
            for ci in range(D // LANES):
                dst_buf[j, pl.ds(ci * LANES, LANES)] = zvec
            return 0
        lax.fori_loop(nvalid, CHUNK, zrow, 0)

    def dst_at(dst_row, g):
        return out_hbm.at[b, pl.ds(_mult8(dst_row + g * CHUNK), CHUNK), :]

    def zero_tail(dst_row, nct, nchunks):
        def zbody(g, _):
            pltpu.make_async_copy(zbuf, dst_at(dst_row, g), zsem).start()
            return 0
        lax.fori_loop(nct, nchunks, zbody, 0)

    def seg_aligned(src_row, dst_row, n, region_rows):
        nfull = n // CHUNK
        rem = n - nfull * CHUNK
        nchunks = region_rows // CHUNK
        nct = nfull + jnp.where(rem > 0, 1, 0)
        zero_tail(dst_row, nct, nchunks)

        def src_at(g):
            return inputs_hbm.at[b, pl.ds(_mult8(src_row + g * CHUNK),
                                          CHUNK), :]

        r0 = ring0.at[pl.ds(0, CHUNK), :]
        r1 = ring1.at[pl.ds(0, CHUNK), :]

        def pair(p, _):
            g0 = 2 * p
            g1 = g0 + 1

            @pl.when(p > 0)
            def _():
                pltpu.make_async_copy(r0, dst_at(dst_row, 0), wsem0).wait()
            pltpu.make_async_copy(src_at(g0), r0, rsem0).start()

            @pl.when(g1 < nfull)
            def _():
                @pl.when(p > 0)
                def _():
                    pltpu.make_async_copy(r1, dst_at(dst_row, 0),
                                          wsem1).wait()
                pltpu.make_async_copy(src_at(g1), r1, rsem1).start()

            pltpu.make_async_copy(src_at(g0), r0, rsem0).wait()
            pltpu.make_async_copy(r0, dst_at(dst_row, g0), wsem0).start()

            @pl.when(g1 < nfull)
            def _():
                pltpu.make_async_copy(src_at(g1), r1, rsem1).wait()
                pltpu.make_async_copy(r1, dst_at(dst_row, g1), wsem1).start()
            return 0
        lax.fori_loop(0, (nfull + 1) // 2, pair, 0)

        @pl.when(nfull >= 1)
        def _():
            pltpu.make_async_copy(r0, dst_at(dst_row, 0), wsem0).wait()

        @pl.when(nfull >= 2)
        def _():
            pltpu.make_async_copy(r1, dst_at(dst_row, 0), wsem1).wait()

        @pl.when(rem > 0)
        def _():
            pltpu.make_async_copy(src_at(nfull), r0, rsem0).start()
            pltpu.make_async_copy(src_at(nfull), r0, rsem0).wait()
            shift_chunk(ring0, 0, rem, obuf0)
            pltpu.make_async_copy(obuf0, dst_at(dst_row, nfull), zsem).start()

        nz = nchunks - nct + jnp.where(rem > 0, 1, 0)

        def zdrain(i, _):
            pltpu.make_async_copy(zbuf, dst_at(dst_row, 0), zsem).wait()
            return 0
        lax.fori_loop(0, nz, zdrain, 0)

    def seg_shifted(src_row, dst_row, n, region_rows):
        nchunks = region_rows // CHUNK
        nct = (n + CHUNK - 1) // CHUNK
        zero_tail(dst_row, nct, nchunks)

        def win(g):
            src = src_row + g * CHUNK
            wstart = _mult8(jnp.minimum(src - src % 8, L1 - WIN))
            return inputs_hbm.at[b, pl.ds(wstart, WIN), :], src - wstart

        def proc(g, ring, obuf, rsem, wsem, first):
            _, sbase = win(g)
            nv = jnp.minimum(n - g * CHUNK, CHUNK)
            pltpu.make_async_copy(win(g)[0], ring, rsem).wait()
            @pl.when(jnp.logical_not(first))
            def _():
                pltpu.make_async_copy(obuf, dst_at(dst_row, 0), wsem).wait()
            shift_chunk(ring, sbase, nv, obuf)
            pltpu.make_async_copy(obuf, dst_at(dst_row, g), wsem).start()

        @pl.when(nct > 0)
        def _():
            pltpu.make_async_copy(win(0)[0], ring0, rsem0).start()

        def pair(p, _):
            g0 = 2 * p
            g1 = g0 + 1

            @pl.when(g1 < nct)
            def _():
                pltpu.make_async_copy(win(g1)[0], ring1, rsem1).start()

            proc(g0, ring0, obuf0, rsem0, wsem0, p == 0)

            @pl.when(g0 + 2 < nct)
            def _():
                pltpu.make_async_copy(win(g0 + 2)[0], ring0, rsem0).start()

            @pl.when(g1 < nct)
            def _():
                proc(g1, ring1, obuf1, rsem1, wsem1, p == 0)
            return 0
        lax.fori_loop(0, (nct + 1) // 2, pair, 0)

        @pl.when(nct >= 1)
        def _():
            pltpu.make_async_copy(obuf0, dst_at(dst_row, 0), wsem0).wait()

        @pl.when(nct >= 2)
        def _():
            pltpu.make_async_copy(obuf1, dst_at(dst_row, 0), wsem1).wait()

        nz = nchunks - nct

        def zdrain(i, _):
            pltpu.make_async_copy(zbuf, dst_at(dst_row, 0), zsem).wait()
            return 0
        lax.fori_loop(0, nz, zdrain, 0)

    q = msum(0, MAXQ)
    t = msum(MAXQ, MAXT)

    @pl.when(half == 0)
    def _():
        seg_aligned(0, 0, q, MAXQ)
        seg_shifted(q, MAXQ, t, MAXT)

    @pl.when(half == 1)
    def _():
        c = msum(MAXQ + MAXT, MAXC)
        seg_shifted(q + t, MAXQ + MAXT, c, MAXC)


@jax.jit
def _repack(inputs, maski):
    mesh = plsc.VectorSubcoreMesh(core_axis_name="c", subcore_axis_name="s")
    k = functools.partial(
        pl.kernel,
        mesh=mesh,
        out_type=jax.ShapeDtypeStruct((B, L2, D), jnp.float32),
        scratch_types=[
            pltpu.VMEM((L2,), jnp.int32),
            pltpu.VMEM((WIN, D), jnp.float32),
            pltpu.VMEM((WIN, D), jnp.float32),
            pltpu.VMEM((CHUNK, D), jnp.float32),
            pltpu.VMEM((CHUNK, D), jnp.float32),
            pltpu.VMEM((CHUNK, D), jnp.float32),
            pltpu.SemaphoreType.DMA,
            pltpu.SemaphoreType.DMA,
            pltpu.SemaphoreType.DMA,
            pltpu.SemaphoreType.DMA,
            pltpu.SemaphoreType.DMA,
            pltpu.SemaphoreType.DMA,
        ],
    )(_repack_body)
    return k(inputs, maski)


def kernel(inputs, mask, mask_split):
    del mask
    outputs = _repack(inputs, mask_split.astype(jnp.int32).reshape(-1))
    return outputs, mask_split


# speedup vs baseline: 11.9408x; 1.0745x over previous
import functools

import jax
import jax.numpy as jnp
from jax import lax
from jax.experimental import pallas as pl
from jax.experimental.pallas import tpu as pltpu
from jax.experimental.pallas import tpu_sc as plsc

B = 16
L1 = 4096
MAXQ = 2048
MAXT = 512
MAXC = 1536
L2 = MAXQ + MAXT + MAXC
D = 512

CHUNK = 32
WIN = CHUNK + 8
LANES = 16


def _mult8(x):
    return pl.multiple_of(x, 8)


def _repack_body(inputs_hbm, maski_hbm, out_hbm,
                 mrow, ring0, ring1, obuf0, obuf1, zbuf,
                 rsem0, rsem1, wsem0, wsem1, zsem, msem):
    cid = lax.axis_index("c")
    sid = lax.axis_index("s")
    wid = sid * 2 + cid
    b = wid // 2
    half = wid % 2

    zvec = jnp.zeros((LANES,), jnp.float32)

    pltpu.make_async_copy(maski_hbm.at[pl.ds(b * L2, L2)], mrow, msem).start()

    def zb(j, _):
        zbuf[j // (D // LANES), pl.ds((j % (D // LANES)) * LANES, LANES)] = zvec
        return 0
    lax.fori_loop(0, (CHUNK * D) // LANES, zb, 0)

    pltpu.make_async_copy(maski_hbm.at[pl.ds(b * L2, L2)], mrow, msem).wait()

    def msum(start, count):
        def sbody(i, acc):
            return acc + mrow[pl.ds(start + i * LANES, LANES)]
        acc = lax.fori_loop(0, count // LANES, sbody,
                            jnp.zeros((LANES,), jnp.int32))
        s = acc[0]
        for i in range(1, LANES):
            s = s + acc[i]
        return s

    def shift_chunk(src_buf, sbase, nvalid, dst_buf):
        def crow(j, _):
            for ci in range(D // LANES):
                dst_buf[j, pl.ds(ci * LANES, LANES)] = (
                    src_buf[sbase + j, pl.ds(ci * LANES, LANES)])
            return 0
        lax.fori_loop(0, nvalid, crow, 0)

        def zrow(j, _):
            for ci in range(D // LANES):
                dst_buf[j, pl.ds(ci * LANES, LANES)] = zvec
            return 0
        lax.fori_loop(nvalid, CHUNK, zrow, 0)

    def dst_at(dst_row, g):
        return out_hbm.at[b, pl.ds(_mult8(dst_row + g * CHUNK), CHUNK), :]

    parity = half

    def zero_tail(dst_row, nct, nchunks):
        kz = (nct - parity + 1) // 2
        kmax = (nchunks - parity + 1) // 2

        def zbody(k, _):
            pltpu.make_async_copy(zbuf, dst_at(dst_row, parity + 2 * k),
                                  zsem).start()
            return 0
        lax.fori_loop(kz, kmax, zbody, 0)
        return kmax - kz

    def seg_aligned(src_row, dst_row, n, region_rows):
        nfull = n // CHUNK
        rem = n - nfull * CHUNK
        nchunks = region_rows // CHUNK
        nct = nfull + jnp.where(rem > 0, 1, 0)
        nzfired = zero_tail(dst_row, nct, nchunks)
        kfull = (nfull - parity + 1) // 2
        own_b = jnp.logical_and(rem > 0, nfull % 2 == parity)

        def src_at(g):
            return inputs_hbm.at[b, pl.ds(_mult8(src_row + g * CHUNK),
                                          CHUNK), :]

        def gm(k):
            return parity + 2 * k

        r0 = ring0.at[pl.ds(0, CHUNK), :]
        r1 = ring1.at[pl.ds(0, CHUNK), :]

        def pair(p, _):
            k0 = 2 * p
            k1 = k0 + 1

            @pl.when(p > 0)
            def _():
                pltpu.make_async_copy(r0, dst_at(dst_row, 0), wsem0).wait()
            pltpu.make_async_copy(src_at(gm(k0)), r0, rsem0).start()

            @pl.when(k1 < kfull)
            def _():
                @pl.when(p > 0)
                def _():
                    pltpu.make_async_copy(r1, dst_at(dst_row, 0),
                                          wsem1).wait()
                pltpu.make_async_copy(src_at(gm(k1)), r1, rsem1).start()

            pltpu.make_async_copy(src_at(0), r0, rsem0).wait()
            pltpu.make_async_copy(r0, dst_at(dst_row, gm(k0)), wsem0).start()

            @pl.when(k1 < kfull)
            def _():
                pltpu.make_async_copy(src_at(0), r1, rsem1).wait()
                pltpu.make_async_copy(r1, dst_at(dst_row, gm(k1)),
                                      wsem1).start()
            return 0
        lax.fori_loop(0, (kfull + 1) // 2, pair, 0)

        @pl.when(kfull >= 1)
        def _():
            pltpu.make_async_copy(r0, dst_at(dst_row, 0), wsem0).wait()

        @pl.when(kfull >= 2)
        def _():
            pltpu.make_async_copy(r1, dst_at(dst_row, 0), wsem1).wait()

        @pl.when(own_b)
        def _():
            pltpu.make_async_copy(src_at(nfull), r0, rsem0).start()
            pltpu.make_async_copy(src_at(0), r0, rsem0).wait()
            shift_chunk(ring0, 0, rem, obuf0)
            pltpu.make_async_copy(obuf0, dst_at(dst_row, nfull), zsem).start()

        nz = nzfired + jnp.where(own_b, 1, 0)

        def zdrain(i, _):
            pltpu.make_async_copy(zbuf, dst_at(dst_row, 0), zsem).wait()
            return 0
        lax.fori_loop(0, nz, zdrain, 0)

    def seg_shifted(src_row, dst_row, n, region_rows):
        nchunks = region_rows // CHUNK
        nct = (n + CHUNK - 1) // CHUNK
        nzfired = zero_tail(dst_row, nct, nchunks)
        kct = (nct - parity + 1) // 2

        def gm(k):
            return parity + 2 * k

        def win(k):
            src = src_row + gm(k) * CHUNK
            wstart = _mult8(jnp.minimum(src - src % 8, L1 - WIN))
            return inputs_hbm.at[b, pl.ds(wstart, WIN), :], src - wstart

        def proc(k, ring, obuf, rsem, wsem, first):
            g = gm(k)
            _, sbase = win(k)
            nv = jnp.minimum(n - g * CHUNK, CHUNK)
            pltpu.make_async_copy(win(0)[0], ring, rsem).wait()
            @pl.when(jnp.logical_not(first))
            def _():
                pltpu.make_async_copy(obuf, dst_at(dst_row, 0), wsem).wait()
            shift_chunk(ring, sbase, nv, obuf)
            pltpu.make_async_copy(obuf, dst_at(dst_row, g), wsem).start()

        @pl.when(kct > 0)
        def _():
            pltpu.make_async_copy(win(0)[0], ring0, rsem0).start()

        def pair(p, _):
            k0 = 2 * p
            k1 = k0 + 1

            @pl.when(k1 < kct)
            def _():
                pltpu.make_async_copy(win(k1)[0], ring1, rsem1).start()

            proc(k0, ring0, obuf0, rsem0, wsem0, p == 0)

            @pl.when(k0 + 2 < kct)
            def _():
                pltpu.make_async_copy(win(k0 + 2)[0], ring0, rsem0).start()

            @pl.when(k1 < kct)
            def _():
                proc(k1, ring1, obuf1, rsem1, wsem1, p == 0)
            return 0
        lax.fori_loop(0, (kct + 1) // 2, pair, 0)

        @pl.when(kct >= 1)
        def _():
            pltpu.make_async_copy(obuf0, dst_at(dst_row, 0), wsem0).wait()

        @pl.when(kct >= 2)
        def _():
            pltpu.make_async_copy(obuf1, dst_at(dst_row, 0), wsem1).wait()

        def zdrain(i, _):
            pltpu.make_async_copy(zbuf, dst_at(dst_row, 0), zsem).wait()
            return 0
        lax.fori_loop(0, nzfired, zdrain, 0)

    q = msum(0, MAXQ)
    t = msum(MAXQ, MAXT)
    c = msum(MAXQ + MAXT, MAXC)

    seg_aligned(0, 0, q, MAXQ)
    seg_shifted(q, MAXQ, t, MAXT)
    seg_shifted(q + t, MAXQ + MAXT, c, MAXC)


@jax.jit
def _repack(inputs, maski):
    mesh = plsc.VectorSubcoreMesh(core_axis_name="c", subcore_axis_name="s")
    k = functools.partial(
        pl.kernel,
        mesh=mesh,
        out_type=jax.ShapeDtypeStruct((B, L2, D), jnp.float32),
        scratch_types=[
            pltpu.VMEM((L2,), jnp.int32),
            pltpu.VMEM((WIN, D), jnp.float32),
            pltpu.VMEM((WIN, D), jnp.float32),
            pltpu.VMEM((CHUNK, D), jnp.float32),
            pltpu.VMEM((CHUNK, D), jnp.float32),
            pltpu.VMEM((CHUNK, D), jnp.float32),
            pltpu.SemaphoreType.DMA,
            pltpu.SemaphoreType.DMA,
            pltpu.SemaphoreType.DMA,
            pltpu.SemaphoreType.DMA,
            pltpu.SemaphoreType.DMA,
            pltpu.SemaphoreType.DMA,
        ],
    )(_repack_body)
    return k(inputs, maski)


def kernel(inputs, mask, mask_split):
    del mask
    outputs = _repack(inputs, mask_split.astype(jnp.int32).reshape(-1))
    return outputs, mask_split
